# baseline (speedup 1.0000x reference)
import jax
import jax.numpy as jnp
from jax.experimental import pallas as pl
from jax.experimental.pallas import tpu as pltpu

B = 4096
IN_DIM = 512
H = 512
E = 8
TOP_K = 2
BT = 1024


def _fused_kernel(x_ref, W_in_ref, b_in_ref, Wg_ref, bg_ref,
                  W1_ref, b1_ref, W2_ref, b2_ref,
                  Wo1_ref, bo1_ref, Wo2_ref, bo2_ref, out_ref):
    x = x_ref[...]
    h = jnp.dot(x, W_in_ref[...], preferred_element_type=jnp.float32)
    h = h + b_in_ref[...]
    gl = jnp.dot(h, Wg_ref[...], preferred_element_type=jnp.float32)
    gl = gl + bg_ref[...]
    gates = jax.nn.softmax(gl, axis=-1)

    eids = jax.lax.broadcasted_iota(jnp.int32, gates.shape, 1)
    m1 = jnp.max(gates, axis=-1)
    i1 = jnp.argmax(gates, axis=-1)
    masked = jnp.where(eids == i1[:, None], -jnp.inf, gates)
    m2 = jnp.max(masked, axis=-1)
    i2 = jnp.argmax(masked, axis=-1)
    denom = m1 + m2
    g1 = m1 / denom
    g2 = m2 / denom

    coefmat = (jnp.where(eids == i1[:, None], g1[:, None], 0.0) +
               jnp.where(eids == i2[:, None], g2[:, None], 0.0))
    acc = jnp.dot(coefmat, b2_ref[...], preferred_element_type=jnp.float32)
    for e in range(E):
        h1 = jnp.dot(h, W1_ref[e], preferred_element_type=jnp.float32)
        h1 = jnp.maximum(h1 + b1_ref[e], 0.0)
        h1 = h1 * coefmat[:, e][:, None]
        acc = acc + jnp.dot(h1, W2_ref[e], preferred_element_type=jnp.float32)

    z = jnp.dot(acc, Wo1_ref[...], preferred_element_type=jnp.float32)
    z = jnp.maximum(z + bo1_ref[...], 0.0)
    p = jnp.dot(z, Wo2_ref[...], preferred_element_type=jnp.float32)
    out_ref[...] = p + bo2_ref[...]


def kernel(x, W_in, b_in, Wg, bg, W1, b1, W2, b2, Wo1, bo1, Wo2, bo2):
    grid = (B // BT,)

    def full(*shape):
        return pl.BlockSpec(shape, lambda i: (0,) * len(shape))

    out = pl.pallas_call(
        _fused_kernel,
        grid=grid,
        in_specs=[
            pl.BlockSpec((BT, IN_DIM), lambda i: (i, 0)),
            full(IN_DIM, H),
            full(1, H),
            full(H, E),
            full(1, E),
            full(E, H, H),
            full(E, H),
            full(E, H, H),
            full(E, H),
            full(H, H // 2),
            full(1, H // 2),
            full(H // 2, 1),
            full(1, 1),
        ],
        out_specs=pl.BlockSpec((BT, 1), lambda i: (i, 0)),
        out_shape=jax.ShapeDtypeStruct((B, 1), jnp.float32),
        compiler_params=pltpu.CompilerParams(
            dimension_semantics=("parallel",),
        ),
    )(x, W_in, b_in.reshape(1, H), Wg, bg.reshape(1, E),
      W1, b1, W2, b2,
      Wo1, bo1.reshape(1, H // 2), Wo2, bo2.reshape(1, 1))
    return out


# speedup vs baseline: 1.0326x; 1.0326x over previous
import jax
import jax.numpy as jnp
from jax.experimental import pallas as pl
from jax.experimental.pallas import tpu as pltpu

B = 4096
IN_DIM = 512
H = 512
E = 8
TOP_K = 2
BT = 2048


def _fused_kernel(x_ref, W_in_ref, b_in_ref, Wg_ref, bg_ref,
                  W1_ref, b1_ref, W2_ref, b2_ref,
                  Wo1_ref, bo1_ref, Wo2_ref, bo2_ref, out_ref):
    x = x_ref[...]
    h = jnp.dot(x, W_in_ref[...], preferred_element_type=jnp.float32)
    h = h + b_in_ref[...]
    gl = jnp.dot(h, Wg_ref[...], preferred_element_type=jnp.float32)
    gl = gl + bg_ref[...]
    gates = jax.nn.softmax(gl, axis=-1)

    eids = jax.lax.broadcasted_iota(jnp.int32, gates.shape, 1)
    m1 = jnp.max(gates, axis=-1)
    i1 = jnp.argmax(gates, axis=-1)
    masked = jnp.where(eids == i1[:, None], -jnp.inf, gates)
    m2 = jnp.max(masked, axis=-1)
    i2 = jnp.argmax(masked, axis=-1)
    denom = m1 + m2
    g1 = m1 / denom
    g2 = m2 / denom

    acc = jnp.zeros((BT, H), jnp.float32)
    for e in range(E):
        h1 = jnp.dot(h, W1_ref[e], preferred_element_type=jnp.float32)
        h1 = jnp.maximum(h1 + b1_ref[e], 0.0)
        o = jnp.dot(h1, W2_ref[e], preferred_element_type=jnp.float32)
        o = o + b2_ref[e]
        coef = jnp.where(i1 == e, g1, 0.0) + jnp.where(i2 == e, g2, 0.0)
        acc = acc + coef[:, None] * o

    z = jnp.dot(acc, Wo1_ref[...], preferred_element_type=jnp.float32)
    z = jnp.maximum(z + bo1_ref[...], 0.0)
    p = jnp.dot(z, Wo2_ref[...], preferred_element_type=jnp.float32)
    out_ref[...] = p + bo2_ref[...]


def kernel(x, W_in, b_in, Wg, bg, W1, b1, W2, b2, Wo1, bo1, Wo2, bo2):
    grid = (B // BT,)

    def full(*shape):
        return pl.BlockSpec(shape, lambda i: (0,) * len(shape))

    out = pl.pallas_call(
        _fused_kernel,
        grid=grid,
        in_specs=[
            pl.BlockSpec((BT, IN_DIM), lambda i: (i, 0)),
            full(IN_DIM, H),
            full(1, H),
            full(H, E),
            full(1, E),
            full(E, H, H),
            full(E, H),
            full(E, H, H),
            full(E, H),
            full(H, H // 2),
            full(1, H // 2),
            full(H // 2, 1),
            full(1, 1),
        ],
        out_specs=pl.BlockSpec((BT, 1), lambda i: (i, 0)),
        out_shape=jax.ShapeDtypeStruct((B, 1), jnp.float32),
        compiler_params=pltpu.CompilerParams(
            dimension_semantics=("parallel",),
        ),
    )(x, W_in, b_in.reshape(1, H), Wg, bg.reshape(1, E),
      W1, b1, W2, b2,
      Wo1, bo1.reshape(1, H // 2), Wo2, bo2.reshape(1, 1))
    return out


# speedup vs baseline: 1.0695x; 1.0358x over previous
import jax
import jax.numpy as jnp
from jax.experimental import pallas as pl
from jax.experimental.pallas import tpu as pltpu

B = 4096
IN_DIM = 512
H = 512
E = 8
TOP_K = 2
BT = 1024


def _fused_kernel(x_ref, W_in_ref, b_in_ref, Wg_ref, bg_ref,
                  W1_ref, b1_ref, W2_ref, b2_ref,
                  Wo1_ref, bo1_ref, Wo2_ref, bo2_ref, out_ref):
    x = x_ref[...]
    h = jnp.dot(x, W_in_ref[...], preferred_element_type=jnp.float32)
    h = h + b_in_ref[...]
    gl = jnp.dot(h, Wg_ref[...], preferred_element_type=jnp.float32)
    gl = gl + bg_ref[...]

    eids = jax.lax.broadcasted_iota(jnp.int32, gl.shape, 1)
    l1 = jnp.max(gl, axis=-1)
    i1 = jnp.argmax(gl, axis=-1)
    masked = jnp.where(eids == i1[:, None], -jnp.inf, gl)
    l2 = jnp.max(masked, axis=-1)
    i2 = jnp.argmax(masked, axis=-1)
    r = jnp.exp(l2 - l1)
    g1 = 1.0 / (1.0 + r)
    g2 = 1.0 - g1

    acc = jnp.zeros((BT, H), jnp.float32)
    for e in range(E):
        h1 = jnp.dot(h, W1_ref[e], preferred_element_type=jnp.float32)
        h1 = jnp.maximum(h1 + b1_ref[e], 0.0)
        o = jnp.dot(h1, W2_ref[e], preferred_element_type=jnp.float32)
        o = o + b2_ref[e]
        coef = jnp.where(i1 == e, g1, 0.0) + jnp.where(i2 == e, g2, 0.0)
        acc = acc + coef[:, None] * o

    z = jnp.dot(acc, Wo1_ref[...], preferred_element_type=jnp.float32)
    z = jnp.maximum(z + bo1_ref[...], 0.0)
    p = jnp.sum(z * Wo2_ref[...], axis=-1, keepdims=True)
    out_ref[...] = p + bo2_ref[...]


def kernel(x, W_in, b_in, Wg, bg, W1, b1, W2, b2, Wo1, bo1, Wo2, bo2):
    grid = (B // BT,)

    def full(*shape):
        return pl.BlockSpec(shape, lambda i: (0,) * len(shape))

    out = pl.pallas_call(
        _fused_kernel,
        grid=grid,
        in_specs=[
            pl.BlockSpec((BT, IN_DIM), lambda i: (i, 0)),
            full(IN_DIM, H),
            full(1, H),
            full(H, E),
            full(1, E),
            full(E, H, H),
            full(E, H),
            full(E, H, H),
            full(E, H),
            full(H, H // 2),
            full(1, H // 2),
            full(1, H // 2),
            full(1, 1),
        ],
        out_specs=pl.BlockSpec((BT, 1), lambda i: (i, 0)),
        out_shape=jax.ShapeDtypeStruct((B, 1), jnp.float32),
        compiler_params=pltpu.CompilerParams(
            dimension_semantics=("parallel",),
        ),
    )(x, W_in, b_in.reshape(1, H), Wg, bg.reshape(1, E),
      W1, b1, W2, b2,
      Wo1, bo1.reshape(1, H // 2), Wo2.reshape(1, H // 2),
      bo2.reshape(1, 1))
    return out
